# baseline (speedup 1.0000x reference)
import functools

import jax
import jax.numpy as jnp
from jax import lax
from jax.experimental import pallas as pl
from jax.experimental.pallas import tpu as pltpu
from jax.experimental.pallas import tpu_sc as plsc

_BB = 128
_HB = 8
_LANES = 16
_OW = 137
_G = 5


def _body(L, H, B, tasks_per_worker, num_cores,
          xt_hbm, emb_hbm, pos_hbm, out_hbm,
          idx_v, pos_v, rows0, rows1, obuf0, obuf1,
          sem_g0, sem_g1, sem_o0, sem_o1):
  wid = lax.axis_index("s") * num_cores + lax.axis_index("c")
  t0 = wid * tasks_per_worker
  n_groups = tasks_per_worker // _G
  grp_rows = _G * _BB

  rows = (rows0, rows1)
  obufs = (obuf0, obuf1)
  gsems = (sem_g0, sem_g1)
  osems = (sem_o0, sem_o1)
  n_tb = B // _BB

  pltpu.sync_copy(xt_hbm.at[pl.ds(t0 * _BB, tasks_per_worker * _BB)], idx_v)
  pltpu.sync_copy(pos_hbm, pos_v)

  lane = jnp.arange(_LANES, dtype=jnp.int32)
  thv = [(lane + 16 * h4) // _HB for h4 in range(H // _LANES)]
  hhv = [lax.rem(lane + 16 * h4, _HB) for h4 in range(H // _LANES)]

  def gather_group(buf_i, g):
    goff = lax.rem(g, n_groups)
    return pltpu.async_copy(
        emb_hbm.at[idx_v.at[pl.ds(goff * grp_rows, grp_rows)]],
        rows[buf_i], gsems[buf_i])

  def wait_gather(buf_i):
    pltpu.make_async_copy(
        emb_hbm.at[idx_v.at[pl.ds(0, grp_rows)]], rows[buf_i],
        gsems[buf_i]).wait()

  def drain_out(o_i):
    pltpu.make_async_copy(
        obufs[o_i].at[:, :, pl.ds(0, _BB)], out_hbm.at[0, :, 0],
        osems[o_i]).wait()

  def transpose_add(buf_i, o_i, i, l):
    rbuf = rows[buf_i]
    obuf = obufs[o_i]
    pv = [pos_v[l, pl.ds(h4 * _LANES, _LANES)] for h4 in range(H // _LANES)]

    def four_b(b4, _):
      b0 = i * _BB + b4 * 4
      for bi in range(4):
        b = b0 + bi
        bs = jnp.full((_LANES,), b4 * 4 + bi, jnp.int32)
        for h4 in range(H // _LANES):
          v = rbuf[b, pl.ds(h4 * _LANES, _LANES)] + pv[h4]
          plsc.store_scatter(obuf, [thv[h4], hhv[h4], bs], v)
      return 0

    lax.fori_loop(0, _BB // 4, four_b, 0)

  def write_out(o_i, l, tb):
    pltpu.async_copy(
        obufs[o_i].at[:, :, pl.ds(0, _BB)], out_hbm.at[l, :, tb],
        osems[o_i])

  gather_group(0, 0)

  def group_pair(gp, _):
    for par in range(2):
      g = gp * 2 + par
      wait_gather(par)
      gather_group(1 - par, g + 1)
      for i in range(_G):
        k = g * _G + i
        t = t0 + k
        l = t // n_tb
        tb = lax.rem(t, n_tb)
        oi = (par * _G + i) % 2

        @pl.when(k >= 2)
        def _():
          drain_out(oi)

        transpose_add(par, oi, i, l)
        write_out(oi, l, tb)
    return 0

  lax.fori_loop(0, n_groups // 2, group_pair, 0)

  wait_gather(0)
  drain_out(0)
  drain_out(1)


def kernel(x, emb_table, pos_table):
  B, L = x.shape
  V, H = emb_table.shape
  info = plsc.get_sparse_core_info()
  nw = info.num_cores * info.num_subcores
  n_tb = B // _BB
  tasks_per_worker = (L * n_tb) // nw

  mesh = plsc.VectorSubcoreMesh(core_axis_name="c", subcore_axis_name="s")
  body = functools.partial(_body, L, H, B, tasks_per_worker, info.num_cores)
  run = pl.kernel(
      body,
      out_type=jax.ShapeDtypeStruct((L, H // _HB, n_tb, _HB, _BB),
                                    jnp.float32),
      mesh=mesh,
      scratch_types=[
          pltpu.VMEM((tasks_per_worker * _BB,), jnp.int32),
          pltpu.VMEM((L, H), jnp.float32),
          pltpu.VMEM((_G * _BB, H), jnp.float32),
          pltpu.VMEM((_G * _BB, H), jnp.float32),
          pltpu.VMEM((H // _HB, _HB, _OW), jnp.float32),
          pltpu.VMEM((H // _HB, _HB, _OW), jnp.float32),
          pltpu.SemaphoreType.DMA,
          pltpu.SemaphoreType.DMA,
          pltpu.SemaphoreType.DMA,
          pltpu.SemaphoreType.DMA,
      ],
      compiler_params=pltpu.CompilerParams(use_tc_tiling_on_sc=False,
                                           needs_layout_passes=False),
  )
  xt = jnp.swapaxes(x, 0, 1).reshape(-1)
  out5 = run(xt, emb_table, pos_table)
  return out5.transpose(2, 4, 0, 1, 3).reshape(B, L, H)


# speedup vs baseline: 1.8849x; 1.8849x over previous
import functools

import jax
import jax.numpy as jnp
from jax import lax
from jax.experimental import pallas as pl
from jax.experimental.pallas import tpu as pltpu
from jax.experimental.pallas import tpu_sc as plsc

_BB = 128
_HB = 8
_LANES = 16
_OW = 137
_G = 5


def _body(L, H, B, tasks_per_worker, num_cores,
          xt_hbm, emb_hbm, pos_hbm, out_hbm,
          idx_v, pos_v, rows0, rows1, obuf0, obuf1,
          sem_g0, sem_g1, sem_o0, sem_o1):
  wid = lax.axis_index("s") * num_cores + lax.axis_index("c")
  t0 = wid * tasks_per_worker
  n_groups = tasks_per_worker // _G
  grp_rows = _G * _BB

  rows = (rows0, rows1)
  obufs = (obuf0, obuf1)
  gsems = (sem_g0, sem_g1)
  osems = (sem_o0, sem_o1)
  n_tb = B // _BB

  pltpu.sync_copy(xt_hbm.at[pl.ds(t0 * _BB, tasks_per_worker * _BB)], idx_v)
  pltpu.sync_copy(pos_hbm, pos_v)

  lane = jnp.arange(_LANES, dtype=jnp.int32)
  thv = [(lane + 16 * h4) // _HB for h4 in range(H // _LANES)]
  hhv = [lax.rem(lane + 16 * h4, _HB) for h4 in range(H // _LANES)]

  def gather_group(buf_i, g):
    goff = lax.rem(g, n_groups)
    return pltpu.async_copy(
        emb_hbm.at[idx_v.at[pl.ds(goff * grp_rows, grp_rows)]],
        rows[buf_i], gsems[buf_i])

  def wait_gather(buf_i):
    pltpu.make_async_copy(
        emb_hbm.at[idx_v.at[pl.ds(0, grp_rows)]], rows[buf_i],
        gsems[buf_i]).wait()

  def drain_out(o_i):
    pltpu.make_async_copy(
        obufs[o_i].at[:, :, pl.ds(0, _BB)], out_hbm.at[0, :, 0],
        osems[o_i]).wait()

  def transpose_add(buf_i, o_i, i, l):
    rbuf = rows[buf_i]
    obuf = obufs[o_i]
    pv = [pos_v[l, pl.ds(h4 * _LANES, _LANES)] for h4 in range(H // _LANES)]

    @plsc.parallel_loop(0, _BB, step=1, unroll=8)
    def _(b):
      bs = jnp.full((_LANES,), b, jnp.int32)
      for h4 in range(H // _LANES):
        v = rbuf[i * _BB + b, pl.ds(h4 * _LANES, _LANES)] + pv[h4]
        plsc.store_scatter(obuf, [thv[h4], hhv[h4], bs], v)

  def write_out(o_i, l, tb):
    pltpu.async_copy(
        obufs[o_i].at[:, :, pl.ds(0, _BB)], out_hbm.at[l, :, tb],
        osems[o_i])

  gather_group(0, 0)

  def group_pair(gp, _):
    for par in range(2):
      g = gp * 2 + par
      wait_gather(par)
      gather_group(1 - par, g + 1)
      for i in range(_G):
        k = g * _G + i
        t = t0 + k
        l = t // n_tb
        tb = lax.rem(t, n_tb)
        oi = (par * _G + i) % 2

        @pl.when(k >= 2)
        def _():
          drain_out(oi)

        transpose_add(par, oi, i, l)
        write_out(oi, l, tb)
    return 0

  lax.fori_loop(0, n_groups // 2, group_pair, 0)

  wait_gather(0)
  drain_out(0)
  drain_out(1)


def kernel(x, emb_table, pos_table):
  B, L = x.shape
  V, H = emb_table.shape
  info = plsc.get_sparse_core_info()
  nw = info.num_cores * info.num_subcores
  n_tb = B // _BB
  tasks_per_worker = (L * n_tb) // nw

  mesh = plsc.VectorSubcoreMesh(core_axis_name="c", subcore_axis_name="s")
  body = functools.partial(_body, L, H, B, tasks_per_worker, info.num_cores)
  run = pl.kernel(
      body,
      out_type=jax.ShapeDtypeStruct((L, H // _HB, n_tb, _HB, _BB),
                                    jnp.float32),
      mesh=mesh,
      scratch_types=[
          pltpu.VMEM((tasks_per_worker * _BB,), jnp.int32),
          pltpu.VMEM((L, H), jnp.float32),
          pltpu.VMEM((_G * _BB, H), jnp.float32),
          pltpu.VMEM((_G * _BB, H), jnp.float32),
          pltpu.VMEM((H // _HB, _HB, _OW), jnp.float32),
          pltpu.VMEM((H // _HB, _HB, _OW), jnp.float32),
          pltpu.SemaphoreType.DMA,
          pltpu.SemaphoreType.DMA,
          pltpu.SemaphoreType.DMA,
          pltpu.SemaphoreType.DMA,
      ],
      compiler_params=pltpu.CompilerParams(use_tc_tiling_on_sc=False,
                                           needs_layout_passes=False),
  )
  xt = jnp.swapaxes(x, 0, 1).reshape(-1)
  out5 = run(xt, emb_table, pos_table)
  return out5.transpose(2, 4, 0, 1, 3).reshape(B, L, H)
